# baseline (speedup 1.0000x reference)
import functools
import math

import jax
import jax.numpy as jnp
import numpy as np
from jax import lax
from jax.experimental import pallas as pl
from jax.experimental.pallas import tpu as pltpu
from jax.experimental.pallas import tpu_sc as plsc

N_HEADS = 16
SEQ = 2048
NUM_BUCKETS = 32
WPAD = 4128
NSHIFT = 16
NC, NS = 2, 16
BLOCKS_PER_W = (N_HEADS * SEQ) // (NC * NS * 8)


def _bucket_indices():
    num_buckets = NUM_BUCKETS
    max_distance = max(SEQ, 2)
    p = np.arange(WPAD)
    n = np.abs(p - (SEQ - 1))
    max_exact = max(1, num_buckets // 2)
    is_small = n < max_exact
    n_float = np.maximum(n.astype(np.float32), np.float32(1.0))
    log_scale = math.log(max_distance / max_exact) if max_distance > max_exact else 1.0
    log_scale = max(log_scale, 1e-06)
    val_if_large = max_exact + (
        np.log(n_float / np.float32(max_exact)) / np.float32(log_scale)
        * np.float32(num_buckets - max_exact)
    ).astype(np.int32)
    val_if_large = np.clip(val_if_large, max_exact, num_buckets - 1)
    return jnp.asarray(np.where(is_small, n, val_if_large).astype(np.int32))


def _sc_body(widx_hbm, table_hbm, out_hbm, widx_v, table_v, w_v, wsh_v,
             buf_v, sem0, sem1, sem2):
    wid = lax.axis_index("c") * NS + lax.axis_index("s")
    head = wid // 2
    half = wid % 2
    row0 = half * (BLOCKS_PER_W * 8)

    pltpu.sync_copy(widx_hbm, widx_v)
    pltpu.sync_copy(table_hbm, table_v)

    iota = lax.iota(jnp.int32, 16)
    hvec = jnp.full((16,), head, dtype=jnp.int32)

    @plsc.parallel_loop(0, WPAD // 16, unroll=8)
    def _build_w(k):
        base = k * 16
        idx = widx_v[pl.ds(base, 16)]
        w_v[pl.ds(base, 16)] = plsc.load_gather(table_v, [idx, hvec])

    for s in range(NSHIFT):
        @plsc.parallel_loop(0, WPAD // 16, unroll=8)
        def _build_shift(k, s=s):
            base = k * 16
            idx = jnp.minimum(iota + (base + s), WPAD - 1)
            wsh_v[pl.ds(s * WPAD + base, 16)] = plsc.load_gather(w_v, [idx])

    def stage(b, p):
        o0 = (SEQ - 1) - (row0 + b * 8)
        for r in range(8):
            o = o0 - r
            s = jnp.bitwise_and(o, NSHIFT - 1)
            f0 = pl.multiple_of(s * WPAD + (o - s), 16)

            @plsc.parallel_loop(0, SEQ // 16, unroll=16)
            def _seg(k, r=r, f0=f0):
                col = k * 16
                buf_v[p, r, pl.ds(col, 16)] = wsh_v[pl.ds(f0 + col, 16)]

    sems = (sem0, sem1, sem2)

    def mk_copy(b, p, sem):
        i0 = pl.multiple_of(row0 + b * 8, 8)
        return pltpu.make_async_copy(
            buf_v.at[p], out_hbm.at[head, pl.ds(i0, 8), :], sem)

    def ring(g, carry):
        for u in range(3):
            b = g * 3 + u

            @pl.when(g > 0)
            def _free_buf(b=b, u=u):
                mk_copy(b - 3, u, sems[u]).wait()

            stage(b, u)

            @pl.when(b < BLOCKS_PER_W)
            def _ship(b=b, u=u):
                mk_copy(b, u, sems[u]).start()
        return carry

    lax.fori_loop(0, (BLOCKS_PER_W + 3) // 3, ring, None)
    mk_copy(BLOCKS_PER_W - 2, 0, sems[0]).wait()
    mk_copy(BLOCKS_PER_W - 1, 1, sems[1]).wait()


def kernel(seq_len, rel_pos_bias_table):
    del seq_len
    widx = _bucket_indices()
    mesh = plsc.VectorSubcoreMesh(core_axis_name="c", subcore_axis_name="s")
    run = functools.partial(
        pl.kernel,
        out_type=jax.ShapeDtypeStruct((N_HEADS, SEQ, SEQ), jnp.float32),
        mesh=mesh,
        compiler_params=pltpu.CompilerParams(needs_layout_passes=False),
        scratch_types=[
            pltpu.VMEM((WPAD,), jnp.int32),
            pltpu.VMEM((NUM_BUCKETS, N_HEADS), jnp.float32),
            pltpu.VMEM((WPAD,), jnp.float32),
            pltpu.VMEM((NSHIFT * WPAD,), jnp.float32),
            pltpu.VMEM((3, 8, SEQ), jnp.float32),
            pltpu.SemaphoreType.DMA,
            pltpu.SemaphoreType.DMA,
            pltpu.SemaphoreType.DMA,
        ],
    )(_sc_body)
    out = run(widx, rel_pos_bias_table)
    return out[None]


# speedup vs baseline: 1.0167x; 1.0167x over previous
import functools
import math

import jax
import jax.numpy as jnp
import numpy as np
from jax import lax
from jax.experimental import pallas as pl
from jax.experimental.pallas import tpu as pltpu
from jax.experimental.pallas import tpu_sc as plsc

N_HEADS = 16
SEQ = 2048
NUM_BUCKETS = 32
WPAD = 4128
NSHIFT = 16
NC, NS = 2, 16
BLOCKS_PER_W = (N_HEADS * SEQ) // (NC * NS * 8)


def _bucket_indices():
    num_buckets = NUM_BUCKETS
    max_distance = max(SEQ, 2)
    p = np.arange(WPAD)
    n = np.abs(p - (SEQ - 1))
    max_exact = max(1, num_buckets // 2)
    is_small = n < max_exact
    n_float = np.maximum(n.astype(np.float32), np.float32(1.0))
    log_scale = math.log(max_distance / max_exact) if max_distance > max_exact else 1.0
    log_scale = max(log_scale, 1e-06)
    val_if_large = max_exact + (
        np.log(n_float / np.float32(max_exact)) / np.float32(log_scale)
        * np.float32(num_buckets - max_exact)
    ).astype(np.int32)
    val_if_large = np.clip(val_if_large, max_exact, num_buckets - 1)
    return jnp.asarray(np.where(is_small, n, val_if_large).astype(np.int32))


def _sc_body(widx_hbm, table_hbm, out_hbm, widx_v, table_v, w_v, wsh_v,
             buf_v, sem0, sem1):
    wid = lax.axis_index("c") * NS + lax.axis_index("s")
    head = wid // 2
    half = wid % 2
    row0 = half * (BLOCKS_PER_W * 8)

    pltpu.sync_copy(widx_hbm, widx_v)
    pltpu.sync_copy(table_hbm, table_v)

    iota = lax.iota(jnp.int32, 16)
    hvec = jnp.full((16,), head, dtype=jnp.int32)

    @plsc.parallel_loop(0, WPAD // 16, unroll=8)
    def _build_w(k):
        base = k * 16
        idx = widx_v[pl.ds(base, 16)]
        w_v[pl.ds(base, 16)] = plsc.load_gather(table_v, [idx, hvec])

    def build_shift(s):
        @plsc.parallel_loop(0, WPAD // 16, unroll=8)
        def _build_shift(k, s=s):
            base = k * 16
            idx = jnp.minimum(iota + (base + s), WPAD - 1)
            wsh_v[pl.ds(s * WPAD + base, 16)] = plsc.load_gather(w_v, [idx])

    for s in range(8, NSHIFT):
        build_shift(s)

    def stage(b, p):
        o0 = (SEQ - 1) - (row0 + b * 8)
        for r in range(8):
            o = o0 - r
            s = jnp.bitwise_and(o, NSHIFT - 1)
            f0 = pl.multiple_of(s * WPAD + (o - s), 16)

            @plsc.parallel_loop(0, SEQ // 16, unroll=16)
            def _seg(k, r=r, f0=f0):
                col = k * 16
                buf_v[p, r, pl.ds(col, 16)] = wsh_v[pl.ds(f0 + col, 16)]

    def mk_copy(b, p, sem):
        i0 = pl.multiple_of(row0 + b * 8, 8)
        return pltpu.make_async_copy(
            buf_v.at[p], out_hbm.at[head, pl.ds(i0, 8), :], sem)

    stage(0, 0)
    mk_copy(0, 0, sem0).start()
    for s in range(8):
        build_shift(s)

    def pair(g2, carry):
        g = g2 * 2

        @pl.when(g2 > 0)
        def _wait_prev_odd():
            mk_copy(g - 1, 1, sem1).wait()

        stage(g + 1, 1)
        mk_copy(g + 1, 1, sem1).start()
        mk_copy(g, 0, sem0).wait()
        stage(g + 2, 0)

        @pl.when(g2 < BLOCKS_PER_W // 2 - 1)
        def _ship_even():
            mk_copy(g + 2, 0, sem0).start()

        return carry

    lax.fori_loop(0, BLOCKS_PER_W // 2, pair, None)
    mk_copy(BLOCKS_PER_W - 1, 1, sem1).wait()


def kernel(seq_len, rel_pos_bias_table):
    del seq_len
    widx = _bucket_indices()
    mesh = plsc.VectorSubcoreMesh(core_axis_name="c", subcore_axis_name="s")
    run = functools.partial(
        pl.kernel,
        out_type=jax.ShapeDtypeStruct((N_HEADS, SEQ, SEQ), jnp.float32),
        mesh=mesh,
        compiler_params=pltpu.CompilerParams(needs_layout_passes=False),
        scratch_types=[
            pltpu.VMEM((WPAD,), jnp.int32),
            pltpu.VMEM((NUM_BUCKETS, N_HEADS), jnp.float32),
            pltpu.VMEM((WPAD,), jnp.float32),
            pltpu.VMEM((NSHIFT * WPAD,), jnp.float32),
            pltpu.VMEM((2, 8, SEQ), jnp.float32),
            pltpu.SemaphoreType.DMA,
            pltpu.SemaphoreType.DMA,
        ],
    )(_sc_body)
    out = run(widx, rel_pos_bias_table)
    return out[None]
